# Initial kernel scaffold; baseline (speedup 1.0000x reference)
import jax
import jax.numpy as jnp
from jax.experimental import pallas as pl


def kernel(h, edge_index, node_type, emb, W1, att_src1, att_dst1, b1, W2, att_src2, att_dst2, b2):
    raise NotImplementedError("write your pallas kernel here")


# speedup vs baseline: 1.7498x; 1.7498x over previous
import functools

import jax
import jax.numpy as jnp
from jax import lax
from jax.experimental import pallas as pl
from jax.experimental.pallas import tpu as pltpu
from jax.experimental.pallas import tpu_sc as plsc

NC = 2
NS = 16
NW = NC * NS
CH = 128
LRELU_SLOPE = 0.2
EPS = 1e-16


def _leaky_relu(z):
    return jnp.where(z > 0, z, LRELU_SLOPE * z)


def _tc1_body(emb_ref, w1_ref, as_ref, ad_ref, xw_ref, es_ref, ed_ref):
    xw = jnp.dot(emb_ref[...], w1_ref[...], preferred_element_type=jnp.float32)
    xw_ref[...] = jnp.concatenate([xw, jnp.zeros((1, xw.shape[1]), jnp.float32)], 0)
    es = jnp.dot(xw, as_ref[...], preferred_element_type=jnp.float32)
    ed = jnp.dot(xw, ad_ref[...], preferred_element_type=jnp.float32)
    es_ref[...] = jnp.concatenate([es, jnp.zeros((1,), jnp.float32)])
    ed_ref[...] = jnp.concatenate([ed, jnp.zeros((1,), jnp.float32)])


def _tc2_body(num_ref, den_ref, b1_ref, w2_ref, as2_ref, ad2_ref,
              xv2_ref, es2_ref, ed2_ref):
    den = jnp.sum(den_ref[...], axis=(0, 1)) + EPS
    num = num_ref[0] + num_ref[1]
    x1 = jax.nn.relu(num / den[:, None] + b1_ref[...][None, :])
    xv2 = jnp.dot(x1, w2_ref[...], preferred_element_type=jnp.float32)
    es2 = jnp.dot(x1, as2_ref[...], preferred_element_type=jnp.float32)
    ed2 = jnp.dot(x1, ad2_ref[...], preferred_element_type=jnp.float32)
    xv2_ref[...] = jnp.concatenate([xv2, jnp.zeros((1,), jnp.float32)])
    es2_ref[...] = jnp.concatenate([es2, jnp.zeros((1,), jnp.float32)])
    ed2_ref[...] = jnp.concatenate([ed2, jnp.zeros((1,), jnp.float32)])


def _tc3_body(num2_ref, den2_ref, out_ref):
    out_ref[...] = (jnp.sum(num2_ref[...], axis=(0, 1))
                    / (jnp.sum(den2_ref[...], axis=(0, 1)) + EPS))


def _sc1_body(n, e_real, e_loop, cpw, sb_chunks,
              srchp_hbm, dsthp_hbm, dstraw_hbm, es_hbm, ed_hbm, xw_hbm,
              z_nh_hbm, z_n_hbm,
              num_out, den_out,
              es_v, ed_v, den_v, shp_v, dhp_v, drw_v, hsrc_v, drw2_v, w_v,
              rows_v, num_sh, sem):
    cid = lax.axis_index("c")
    sid = lax.axis_index("s")
    wid = sid * NC + cid
    spt = (n // NS) // 8 * 8
    rem = n - spt * NS

    pltpu.sync_copy(es_hbm, es_v)
    pltpu.sync_copy(ed_hbm, ed_v)
    pltpu.sync_copy(z_n_hbm, den_v)
    pltpu.sync_copy(z_nh_hbm.at[pl.ds(sid * spt, spt)],
                    num_sh.at[pl.ds(sid * spt, spt)])
    if rem:
        @pl.when(sid == NS - 1)
        def _():
            pltpu.sync_copy(z_nh_hbm.at[pl.ds(spt * NS, rem)],
                            num_sh.at[pl.ds(spt * NS, rem)])
    plsc.subcore_barrier()

    lane = lax.iota(jnp.int32, 16)
    zero16 = jnp.zeros((16,), jnp.int32)

    def superchunk(si, carry):
        c0 = si * sb_chunks
        pltpu.sync_copy(srchp_hbm.at[wid, si], shp_v)
        pltpu.sync_copy(dsthp_hbm.at[wid, si], dhp_v)
        pltpu.sync_copy(dstraw_hbm.at[wid, si], drw_v)

        def chunk(c, carry2):
            def wpass(b, carry3):
                shp = shp_v[c, pl.ds(b * 16, 16)]
                dhp = dhp_v[c, pl.ds(b * 16, 16)]
                d16 = drw_v[c, pl.ds(b * 16, 16)]
                hs = shp & 0xFFFF
                hd = dhp & 0xFFFF
                ess = plsc.load_gather(es_v, [hs])
                edd = plsc.load_gather(ed_v, [hd])
                pos = (wid * cpw + c0 + c) * CH + b * 16 + lane
                keep = ((shp < 0x10000) & (dhp < 0x10000)) | (
                    (pos >= e_real) & (pos < e_loop))
                z = _leaky_relu(ess + edd)
                w = jnp.where(keep, jnp.exp(z), 0.0)
                plsc.addupdate_scatter(den_v, [zero16, d16], w)
                w_v[pl.ds(b * 16, 16)] = w
                hsrc_v[b // 4, pl.ds((b % 4) * 16, 16)] = hs
                drw2_v[b // 4, pl.ds((b % 4) * 16, 16)] = d16
                return carry3

            lax.fori_loop(0, CH // 16, wpass, 0)

            for half in range(2):
                pltpu.async_copy(xw_hbm.at[hsrc_v.at[half]], rows_v, sem).wait()

                def scale(q, carry3, half=half):
                    for j in range(16):
                        e_idx = q * 16 + j
                        wj = plsc.load_gather(
                            w_v, [zero16 + (half * 64 + e_idx)])
                        for c2 in range(8):
                            rows_v[e_idx, pl.ds(c2 * 16, 16)] = (
                                rows_v[e_idx, pl.ds(c2 * 16, 16)] * wj)
                    return carry3

                lax.fori_loop(0, 4, scale, 0)
                pltpu.sync_copy(rows_v, num_sh.at[drw2_v.at[half]], add=True)
            return carry2

        lax.fori_loop(0, sb_chunks, chunk, 0)
        return carry

    lax.fori_loop(0, cpw // sb_chunks, superchunk, 0)
    plsc.subcore_barrier()
    pltpu.sync_copy(num_sh.at[pl.ds(sid * spt, spt)],
                    num_out.at[cid, pl.ds(sid * spt, spt)])
    if rem:
        @pl.when(sid == NS - 1)
        def _():
            pltpu.sync_copy(num_sh.at[pl.ds(spt * NS, rem)],
                            num_out.at[cid, pl.ds(spt * NS, rem)])
    pltpu.sync_copy(den_v, den_out.at[wid])


def _sc2_body(n, e_real, e_loop, cpw, sb_chunks,
              srcp_hbm, dstp_hbm, es_hbm, ed_hbm, xv_hbm, z_n_hbm,
              num_out, den_out,
              es_v, ed_v, xv_v, den_v, num_v, sp_v, dp_v):
    cid = lax.axis_index("c")
    sid = lax.axis_index("s")
    wid = sid * NC + cid

    pltpu.sync_copy(es_hbm, es_v)
    pltpu.sync_copy(ed_hbm, ed_v)
    pltpu.sync_copy(xv_hbm, xv_v)
    pltpu.sync_copy(z_n_hbm, den_v)
    pltpu.sync_copy(z_n_hbm, num_v)

    lane = lax.iota(jnp.int32, 16)
    zero16 = jnp.zeros((16,), jnp.int32)

    def superchunk(si, carry):
        c0 = si * sb_chunks
        pltpu.sync_copy(srcp_hbm.at[wid, si], sp_v)
        pltpu.sync_copy(dstp_hbm.at[wid, si], dp_v)
        for c in range(sb_chunks):
            for b in range(CH // 16):
                sp = sp_v[c, pl.ds(b * 16, 16)]
                dp = dp_v[c, pl.ds(b * 16, 16)]
                s16 = sp & 0xFFFF
                d16 = dp & 0xFFFF
                ess = plsc.load_gather(es_v, [s16])
                edd = plsc.load_gather(ed_v, [d16])
                xvs = plsc.load_gather(xv_v, [s16])
                pos = (wid * cpw + c0 + c) * CH + b * 16 + lane
                keep = ((sp < 0x10000) & (dp < 0x10000)) | (
                    (pos >= e_real) & (pos < e_loop))
                z = _leaky_relu(ess + edd)
                w = jnp.where(keep, jnp.exp(z), 0.0)
                plsc.addupdate_scatter(den_v, [zero16, d16], w)
                plsc.addupdate_scatter(num_v, [zero16, d16], w * xvs)
        return carry

    lax.fori_loop(0, cpw // sb_chunks, superchunk, 0)
    pltpu.sync_copy(den_v, den_out.at[wid])
    pltpu.sync_copy(num_v, num_out.at[wid])


def kernel(h, edge_index, node_type, emb, W1, att_src1, att_dst1, b1,
           W2, att_src2, att_dst2, b2):
    n, hdim = emb.shape
    e = edge_index.shape[1]
    e_loop = e + n
    sb = 9
    cpw0 = -(-e_loop // (NW * CH))
    cpw = -(-cpw0 // sb) * sb
    epw = cpw * CH
    ep = epw * NW

    i32 = jnp.int32
    loops = jnp.arange(n, dtype=i32)
    pad = ep - e_loop
    src_ext = jnp.concatenate([edge_index[0].astype(i32), loops,
                               jnp.full((pad,), n, i32)])
    dst_ext = jnp.concatenate([edge_index[1].astype(i32), loops,
                               jnp.zeros((pad,), i32)])
    h_ext = jnp.concatenate([h.astype(i32), jnp.array([n], i32)])
    nt_ext = jnp.concatenate([node_type.astype(i32), jnp.array([1], i32)])
    nt_s = nt_ext[src_ext] << 16
    nt_d = nt_ext[dst_ext] << 16
    eshape = (NW, cpw // sb, sb, CH)
    src_hp = (h_ext[src_ext] | nt_s).reshape(eshape)
    dst_hp = (h_ext[dst_ext] | nt_d).reshape(eshape)
    dst_raw = dst_ext.reshape(eshape)
    src_p2 = (src_ext | nt_s).reshape(eshape)
    dst_p2 = (dst_ext | nt_d).reshape(eshape)
    z_nh = jnp.zeros((n, hdim), jnp.float32)
    z_n = jnp.zeros((1, n), jnp.float32)

    f32 = jnp.float32
    sds = jax.ShapeDtypeStruct

    xw_ext, es_ext, ed_ext = pl.pallas_call(
        _tc1_body,
        out_shape=[sds((n + 1, hdim), f32), sds((n + 1,), f32), sds((n + 1,), f32)],
    )(emb, W1, att_src1, att_dst1)

    mesh = plsc.VectorSubcoreMesh(core_axis_name="c", subcore_axis_name="s",
                                  num_cores=NC, num_subcores=NS)
    sc_params = pltpu.CompilerParams(needs_layout_passes=False)
    sc1 = pl.kernel(
        functools.partial(_sc1_body, n, e, e_loop, cpw, sb),
        out_type=[sds((NC, n, hdim), f32), sds((NW, 1, n), f32)],
        mesh=mesh,
        compiler_params=sc_params,
        scratch_types=[
            pltpu.VMEM((n + 1,), f32),
            pltpu.VMEM((n + 1,), f32),
            pltpu.VMEM((1, n), f32),
            pltpu.VMEM((sb, CH), i32),
            pltpu.VMEM((sb, CH), i32),
            pltpu.VMEM((sb, CH), i32),
            pltpu.VMEM((2, CH // 2), i32),
            pltpu.VMEM((2, CH // 2), i32),
            pltpu.VMEM((CH,), f32),
            pltpu.VMEM((CH // 2, hdim), f32),
            pltpu.VMEM_SHARED((n, hdim), f32),
            pltpu.SemaphoreType.DMA,
        ],
    )
    num1, den1 = sc1(src_hp, dst_hp, dst_raw, es_ext, ed_ext, xw_ext, z_nh, z_n)

    w2v = W2[:, 0]
    as2v = w2v * att_src2[0]
    ad2v = w2v * att_dst2[0]
    xv2_ext, es2_ext, ed2_ext = pl.pallas_call(
        _tc2_body,
        out_shape=[sds((n + 1,), f32), sds((n + 1,), f32), sds((n + 1,), f32)],
    )(num1, den1, b1, w2v, as2v, ad2v)

    sc2 = pl.kernel(
        functools.partial(_sc2_body, n, e, e_loop, cpw, sb),
        out_type=[sds((NW, 1, n), f32), sds((NW, 1, n), f32)],
        mesh=mesh,
        compiler_params=sc_params,
        scratch_types=[
            pltpu.VMEM((n + 1,), f32),
            pltpu.VMEM((n + 1,), f32),
            pltpu.VMEM((n + 1,), f32),
            pltpu.VMEM((1, n), f32),
            pltpu.VMEM((1, n), f32),
            pltpu.VMEM((sb, CH), i32),
            pltpu.VMEM((sb, CH), i32),
        ],
    )
    num2, den2 = sc2(src_p2, dst_p2, es2_ext, ed2_ext, xv2_ext, z_n)

    out = pl.pallas_call(
        _tc3_body, out_shape=sds((n,), f32),
    )(num2, den2)
    return (out + b2[0])[:, None]


# speedup vs baseline: 36.4536x; 20.8335x over previous
import functools

import jax
import jax.numpy as jnp
from jax import lax
from jax.experimental import pallas as pl
from jax.experimental.pallas import tpu as pltpu
from jax.experimental.pallas import tpu_sc as plsc

NC = 2
NS = 16
NW = NC * NS
CH = 128
RB = 48
LRELU_SLOPE = 0.2
EPS = 1e-16


def _leaky_relu(z):
    return jnp.where(z > 0, z, LRELU_SLOPE * z)


def _tc1_body(emb_ref, w1_ref, as_ref, ad_ref, xw_ref, es_ref, ed_ref):
    xw = jnp.dot(emb_ref[...], w1_ref[...], preferred_element_type=jnp.float32)
    xw_ref[...] = jnp.concatenate([xw, jnp.zeros((1, xw.shape[1]), jnp.float32)], 0)
    es = jnp.dot(xw, as_ref[...], preferred_element_type=jnp.float32)
    ed = jnp.dot(xw, ad_ref[...], preferred_element_type=jnp.float32)
    es_ref[...] = jnp.concatenate([es, jnp.zeros((1,), jnp.float32)])
    ed_ref[...] = jnp.concatenate([ed, jnp.zeros((1,), jnp.float32)])


def _tc2_body(num_ref, den_ref, b1_ref, w2_ref, as2_ref, ad2_ref,
              xv2_ref, es2_ref, ed2_ref):
    den = jnp.sum(den_ref[...], axis=(0, 1)) + EPS
    num = num_ref[0] + num_ref[1]
    x1 = jax.nn.relu(num / den[:, None] + b1_ref[...][None, :])
    xv2 = jnp.dot(x1, w2_ref[...], preferred_element_type=jnp.float32)
    es2 = jnp.dot(x1, as2_ref[...], preferred_element_type=jnp.float32)
    ed2 = jnp.dot(x1, ad2_ref[...], preferred_element_type=jnp.float32)
    xv2_ref[...] = jnp.concatenate([xv2, jnp.zeros((1,), jnp.float32)])
    es2_ref[...] = jnp.concatenate([es2, jnp.zeros((1,), jnp.float32)])
    ed2_ref[...] = jnp.concatenate([ed2, jnp.zeros((1,), jnp.float32)])


def _tc3_body(num2_ref, den2_ref, out_ref):
    out_ref[...] = (jnp.sum(num2_ref[...], axis=(0, 1))
                    / (jnp.sum(den2_ref[...], axis=(0, 1)) + EPS))


def _sc1_body(n, e_real, e_loop, cpw, sb_chunks,
              src_hbm, dst_hbm, hn_hbm, es_hbm, ed_hbm, xw_hbm,
              z_nh_hbm, z_n_hbm,
              num_out, den_out,
              hn_v, es_v, ed_v, den_v, s_v, d_v, hsc_v, drc_v, wc_v,
              drwfix_v, rows_v, num_sh, sem):
    cid = lax.axis_index("c")
    sid = lax.axis_index("s")
    wid = sid * NC + cid
    spt = (n // NS) // 8 * 8
    rem = n - spt * NS

    pltpu.sync_copy(hn_hbm, hn_v)
    pltpu.sync_copy(es_hbm, es_v)
    pltpu.sync_copy(ed_hbm, ed_v)
    pltpu.sync_copy(z_n_hbm, den_v)
    pltpu.sync_copy(z_nh_hbm.at[pl.ds(sid * spt, spt)],
                    num_sh.at[pl.ds(sid * spt, spt)])
    if rem:
        @pl.when(sid == NS - 1)
        def _():
            pltpu.sync_copy(z_nh_hbm.at[pl.ds(spt * NS, rem)],
                            num_sh.at[pl.ds(spt * NS, rem)])
    plsc.subcore_barrier()

    lane = lax.iota(jnp.int32, 16)
    zero16 = jnp.zeros((16,), jnp.int32)

    def bufinit(i, carry):
        hsc_v[pl.ds(i * 16, 16)] = zero16
        drc_v[pl.ds(i * 16, 16)] = zero16
        return carry

    lax.fori_loop(0, (sb_chunks * CH + RB) // 16, bufinit, 0)

    def superchunk(si, carry):
        c0 = si * sb_chunks
        pltpu.sync_copy(src_hbm.at[wid, si], s_v)
        pltpu.sync_copy(dst_hbm.at[wid, si], d_v)

        def wpass(g, cur):
            c = g // (CH // 16)
            b = g % (CH // 16)
            s16 = s_v[c, pl.ds(b * 16, 16)]
            d16 = d_v[c, pl.ds(b * 16, 16)]
            shp = plsc.load_gather(hn_v, [s16])
            dhp = plsc.load_gather(hn_v, [d16])
            hs = shp & 0xFFFF
            hd = dhp & 0xFFFF
            ess = plsc.load_gather(es_v, [hs])
            edd = plsc.load_gather(ed_v, [hd])
            pos = (wid * cpw + c0 + c) * CH + b * 16 + lane
            keep = ((shp < 0x10000) & (dhp < 0x10000)) | (
                (pos >= e_real) & (pos < e_loop))
            z = _leaky_relu(ess + edd)
            w = jnp.where(keep, jnp.exp(z), 0.0)
            plsc.addupdate_scatter(den_v, [zero16, d16], w)
            plsc.store_compressed(hsc_v.at[pl.ds(cur, 16)], hs, mask=keep)
            plsc.store_compressed(drc_v.at[pl.ds(cur, 16)], d16, mask=keep)
            plsc.store_compressed(wc_v.at[pl.ds(cur, 16)], w, mask=keep)
            return cur + jnp.sum(keep.astype(jnp.int32))

        cnt = lax.fori_loop(0, sb_chunks * (CH // 16), wpass, jnp.int32(0))
        zf16 = jnp.zeros((16,), jnp.float32)
        for t in range(RB // 16):
            wc_v[pl.ds(cnt + t * 16, 16)] = zf16

        def rowbatch(k, carry2):
            base = k * RB
            for t in range(RB // 16):
                drwfix_v[0, pl.ds(t * 16, 16)] = drc_v[pl.ds(base + t * 16, 16)]
            pltpu.async_copy(
                xw_hbm.at[hsc_v.at[pl.ds(base, RB)]], rows_v, sem).wait()

            def scale(q, carry3):
                for j in range(16):
                    e_idx = q * 16 + j
                    wj = plsc.load_gather(wc_v, [zero16 + (base + e_idx)])
                    for c2 in range(8):
                        rows_v[e_idx, pl.ds(c2 * 16, 16)] = (
                            rows_v[e_idx, pl.ds(c2 * 16, 16)] * wj)
                return carry3

            lax.fori_loop(0, RB // 16, scale, 0)
            pltpu.sync_copy(rows_v, num_sh.at[drwfix_v.at[0]], add=True)
            return carry2

        lax.fori_loop(0, (cnt + RB - 1) // RB, rowbatch, 0)
        return carry

    lax.fori_loop(0, cpw // sb_chunks, superchunk, 0)
    plsc.subcore_barrier()
    pltpu.sync_copy(num_sh.at[pl.ds(sid * spt, spt)],
                    num_out.at[cid, pl.ds(sid * spt, spt)])
    if rem:
        @pl.when(sid == NS - 1)
        def _():
            pltpu.sync_copy(num_sh.at[pl.ds(spt * NS, rem)],
                            num_out.at[cid, pl.ds(spt * NS, rem)])
    pltpu.sync_copy(den_v, den_out.at[wid])


def _sc2_body(n, e_real, e_loop, cpw, sb_chunks,
              src_hbm, dst_hbm, nt_hbm, es_hbm, ed_hbm, xv_hbm, z_n_hbm,
              num_out, den_out,
              nt_v, es_v, ed_v, xv_v, den_v, num_v, s_v, d_v):
    cid = lax.axis_index("c")
    sid = lax.axis_index("s")
    wid = sid * NC + cid

    pltpu.sync_copy(nt_hbm, nt_v)
    pltpu.sync_copy(es_hbm, es_v)
    pltpu.sync_copy(ed_hbm, ed_v)
    pltpu.sync_copy(xv_hbm, xv_v)
    pltpu.sync_copy(z_n_hbm, den_v)
    pltpu.sync_copy(z_n_hbm, num_v)

    lane = lax.iota(jnp.int32, 16)
    zero16 = jnp.zeros((16,), jnp.int32)

    def superchunk(si, carry):
        c0 = si * sb_chunks
        pltpu.sync_copy(src_hbm.at[wid, si], s_v)
        pltpu.sync_copy(dst_hbm.at[wid, si], d_v)

        def wpass(g, carry2):
            c = g // (CH // 16)
            b = g % (CH // 16)
            s16 = s_v[c, pl.ds(b * 16, 16)]
            d16 = d_v[c, pl.ds(b * 16, 16)]
            nts = plsc.load_gather(nt_v, [s16])
            ntd = plsc.load_gather(nt_v, [d16])
            ess = plsc.load_gather(es_v, [s16])
            edd = plsc.load_gather(ed_v, [d16])
            xvs = plsc.load_gather(xv_v, [s16])
            pos = (wid * cpw + c0 + c) * CH + b * 16 + lane
            keep = ((nts == 0) & (ntd == 0)) | (
                (pos >= e_real) & (pos < e_loop))
            z = _leaky_relu(ess + edd)
            w = jnp.where(keep, jnp.exp(z), 0.0)
            plsc.addupdate_scatter(den_v, [zero16, d16], w)
            plsc.addupdate_scatter(num_v, [zero16, d16], w * xvs)
            return carry2

        lax.fori_loop(0, sb_chunks * (CH // 16), wpass, 0)
        return carry

    lax.fori_loop(0, cpw // sb_chunks, superchunk, 0)
    pltpu.sync_copy(den_v, den_out.at[wid])
    pltpu.sync_copy(num_v, num_out.at[wid])


def kernel(h, edge_index, node_type, emb, W1, att_src1, att_dst1, b1,
           W2, att_src2, att_dst2, b2):
    n, hdim = emb.shape
    e = edge_index.shape[1]
    e_loop = e + n
    sb = 3
    cpw0 = -(-e_loop // (NW * CH))
    cpw = -(-cpw0 // sb) * sb
    epw = cpw * CH
    ep = epw * NW

    i32 = jnp.int32
    loops = jnp.arange(n, dtype=i32)
    pad = ep - e_loop
    eshape = (NW, cpw // sb, sb, CH)
    src_4d = jnp.concatenate([edge_index[0].astype(i32), loops,
                              jnp.full((pad,), n, i32)]).reshape(eshape)
    dst_4d = jnp.concatenate([edge_index[1].astype(i32), loops,
                              jnp.zeros((pad,), i32)]).reshape(eshape)
    hn_ext = jnp.concatenate([h.astype(i32) | (node_type.astype(i32) << 16),
                              jnp.array([n | (1 << 16)], i32)])
    nt_ext = jnp.concatenate([node_type.astype(i32), jnp.array([1], i32)])
    z_nh = jnp.zeros((n, hdim), jnp.float32)
    z_n = jnp.zeros((1, n), jnp.float32)

    f32 = jnp.float32
    sds = jax.ShapeDtypeStruct

    xw_ext, es_ext, ed_ext = pl.pallas_call(
        _tc1_body,
        out_shape=[sds((n + 1, hdim), f32), sds((n + 1,), f32), sds((n + 1,), f32)],
    )(emb, W1, att_src1, att_dst1)

    mesh = plsc.VectorSubcoreMesh(core_axis_name="c", subcore_axis_name="s",
                                  num_cores=NC, num_subcores=NS)
    sc_params = pltpu.CompilerParams(needs_layout_passes=False)
    sc1 = pl.kernel(
        functools.partial(_sc1_body, n, e, e_loop, cpw, sb),
        out_type=[sds((NC, n, hdim), f32), sds((NW, 1, n), f32)],
        mesh=mesh,
        compiler_params=sc_params,
        scratch_types=[
            pltpu.VMEM((n + 1,), i32),
            pltpu.VMEM((n + 1,), f32),
            pltpu.VMEM((n + 1,), f32),
            pltpu.VMEM((1, n), f32),
            pltpu.VMEM((sb, CH), i32),
            pltpu.VMEM((sb, CH), i32),
            pltpu.VMEM((sb * CH + RB,), i32),
            pltpu.VMEM((sb * CH + RB,), i32),
            pltpu.VMEM((sb * CH + RB,), f32),
            pltpu.VMEM((1, RB), i32),
            pltpu.VMEM((RB, hdim), f32),
            pltpu.VMEM_SHARED((n, hdim), f32),
            pltpu.SemaphoreType.DMA,
        ],
    )
    num1, den1 = sc1(src_4d, dst_4d, hn_ext, es_ext, ed_ext, xw_ext, z_nh, z_n)

    w2v = W2[:, 0]
    as2v = w2v * att_src2[0]
    ad2v = w2v * att_dst2[0]
    xv2_ext, es2_ext, ed2_ext = pl.pallas_call(
        _tc2_body,
        out_shape=[sds((n + 1,), f32), sds((n + 1,), f32), sds((n + 1,), f32)],
    )(num1, den1, b1, w2v, as2v, ad2v)

    sc2 = pl.kernel(
        functools.partial(_sc2_body, n, e, e_loop, cpw, sb),
        out_type=[sds((NW, 1, n), f32), sds((NW, 1, n), f32)],
        mesh=mesh,
        compiler_params=sc_params,
        scratch_types=[
            pltpu.VMEM((n + 1,), i32),
            pltpu.VMEM((n + 1,), f32),
            pltpu.VMEM((n + 1,), f32),
            pltpu.VMEM((n + 1,), f32),
            pltpu.VMEM((1, n), f32),
            pltpu.VMEM((1, n), f32),
            pltpu.VMEM((sb, CH), i32),
            pltpu.VMEM((sb, CH), i32),
        ],
    )
    num2, den2 = sc2(src_4d, dst_4d, nt_ext, es2_ext, ed2_ext, xv2_ext, z_n)

    out = pl.pallas_call(
        _tc3_body, out_shape=sds((n,), f32),
    )(num2, den2)
    return (out + b2[0])[:, None]
